# baseline (speedup 1.0000x reference)
import functools

import jax
import jax.numpy as jnp
from jax import lax
from jax.experimental import pallas as pl
from jax.experimental.pallas import tpu as pltpu
from jax.experimental.pallas import tpu_sc as plsc

D_MODEL = 64
SCALE = 8.0

_NC, _NS = 2, 16
_NW = _NC * _NS
_B = 4096 * 200
_IDX_W = 128
_K = 8
_CHUNK = _K * _IDX_W
_PER_W = _B // _NW
_NCHUNK = _PER_W // _CHUNK


def _emb_body(x_hbm, table_hbm, out_hbm, idx_v, rows_v, gsem):
    wid = lax.axis_index("s") * _NC + lax.axis_index("c")
    row_base = pl.multiple_of(wid * _PER_W, _CHUNK)
    blk_base = row_base // _IDX_W

    @pl.loop(0, _NCHUNK)
    def _chunk(i):
        blk_off = pl.multiple_of(blk_base + i * _K, _K)
        pltpu.sync_copy(x_hbm.at[pl.ds(blk_off, _K)], idx_v)
        copies = []
        for j in range(_K):
            copies.append(
                pltpu.async_copy(
                    table_hbm.at[idx_v.at[j]],
                    rows_v.at[pl.ds(j * _IDX_W, _IDX_W)],
                    gsem,
                )
            )
        for c in copies:
            c.wait()

        @pl.loop(0, _CHUNK, unroll=2)
        def _row(r):
            for g in range(D_MODEL // 16):
                sl = pl.ds(g * 16, 16)
                rows_v[r, sl] = rows_v[r, sl] * SCALE

        row_off = pl.multiple_of(row_base + i * _CHUNK, _CHUNK)
        pltpu.sync_copy(rows_v, out_hbm.at[pl.ds(row_off, _CHUNK)])


@jax.jit
def _emb(xf, table):
    mesh = plsc.VectorSubcoreMesh(
        core_axis_name="c", subcore_axis_name="s",
        num_cores=_NC, num_subcores=_NS,
    )
    f = pl.kernel(
        _emb_body,
        out_type=jax.ShapeDtypeStruct((_B, D_MODEL), jnp.float32),
        mesh=mesh,
        scratch_types=[
            pltpu.VMEM((_K, _IDX_W), jnp.int32),
            pltpu.VMEM((_CHUNK, D_MODEL), jnp.float32),
            pltpu.SemaphoreType.DMA,
        ],
        compiler_params=pltpu.CompilerParams(use_tc_tiling_on_sc=False),
    )
    return f(xf, table)


def kernel(x, table):
    xf = x.reshape(_B // _IDX_W, _IDX_W)
    out = _emb(xf, table)
    return out.reshape(x.shape + (D_MODEL,))
